# Initial kernel scaffold; baseline (speedup 1.0000x reference)
import jax
import jax.numpy as jnp
from jax.experimental import pallas as pl


def kernel(pcd, c_input, edge_index, W_flat, b_flat, W_fc, b_fc, Wg1, bg1, Wg2, bg2, Wg3, bg3):
    raise NotImplementedError("write your pallas kernel here")


# speedup vs baseline: 3.2047x; 3.2047x over previous
import functools

import jax
import jax.numpy as jnp
from jax import lax
from jax.experimental import pallas as pl
from jax.experimental.pallas import tpu as pltpu
from jax.experimental.pallas import tpu_sc as plsc

_BS, _NP, _K, _DIN, _D, _E = 4, 2500, 8, 16, 128, 320000
_N = _BS * _NP
_NC, _NS = 2, 16
_NW = _NC * _NS
_BLK = 128
_BPT = 80
_EPAD = _NW * _BPT * _BLK
_NBLK = _NW * _BPT
_RPT = 640
_ACC_ROWS = _NS * _RPT
_ZR = 64


def _tc_pre(pcd, c2, W_flat, b_flat, W_fc, b_fc):
    def body(pcd_ref, c2_ref, wf_ref, bf_ref, wfc_ref, bfc_ref, o_ref):
        wbig = jnp.tile(wf_ref[...], (_K, 1)) * (1.0 / _K)
        flat = jnp.dot(c2_ref[...], wbig,
                       preferred_element_type=jnp.float32) + bf_ref[...]
        pcd_v = pcd_ref[...]
        cic = (pcd_v - jnp.mean(pcd_v, axis=1, keepdims=True) - 1.0)
        cic = cic.reshape(_N, 2)
        x = jnp.dot(flat, wfc_ref[0:_D, :], preferred_element_type=jnp.float32)
        x = x + cic[:, 0:1] * wfc_ref[_D:_D + 1, :]
        x = x + cic[:, 1:2] * wfc_ref[_D + 1:_D + 2, :]
        o_ref[...] = x + bfc_ref[...]

    return pl.pallas_call(
        body,
        out_shape=jax.ShapeDtypeStruct((_N, _D), jnp.float32),
    )(pcd, c2, W_flat, b_flat, W_fc, b_fc)


def _sc_segment_sum(h, idx3d):
    mesh = plsc.VectorSubcoreMesh(core_axis_name="c", subcore_axis_name="s")

    @functools.partial(
        pl.kernel,
        out_type=jax.ShapeDtypeStruct((_NC, _ACC_ROWS, _D), jnp.float32),
        mesh=mesh,
        scratch_types=[
            pltpu.VMEM((2, _BLK), jnp.int32),
            pltpu.VMEM((2, _BLK), jnp.int32),
            pltpu.VMEM((2, _BLK), jnp.int32),
            pltpu.VMEM((2, _BLK), jnp.int32),
            pltpu.VMEM((_BLK, _D), jnp.float32),
            pltpu.VMEM((_BLK, _D), jnp.float32),
            pltpu.VMEM((_ZR, _D), jnp.float32),
            pltpu.VMEM_SHARED((_ACC_ROWS, _D), jnp.float32),
            pltpu.SemaphoreType.DMA,
            pltpu.SemaphoreType.DMA,
            pltpu.SemaphoreType.DMA,
            pltpu.SemaphoreType.DMA,
            pltpu.SemaphoreType.DMA,
            pltpu.SemaphoreType.DMA,
        ],
    )
    def k(h_hbm, idx_hbm, out_hbm,
          ib0, ib1, ib2, ib3, gb0, gb1, zbuf, acc,
          is0, is1, is2, is3, gs0, gs1):
        cid = lax.axis_index("c")
        sid = lax.axis_index("s")
        wid = cid * _NS + sid
        base = wid * _BPT
        ibufs = (ib0, ib1, ib2, ib3)
        isems = (is0, is1, is2, is3)
        gbufs = (gb0, gb1)
        gsems = (gs0, gs1)

        zv = jnp.zeros((16,), jnp.float32)

        @pl.loop(0, _ZR)
        def _(i):
            @pl.loop(0, _D, step=16)
            def _(j):
                zbuf.at[i, pl.ds(j, 16)][...] = zv

        @pl.loop(0, _RPT // _ZR)
        def _(z):
            pltpu.sync_copy(zbuf, acc.at[pl.ds(sid * _RPT + z * _ZR, _ZR)])

        plsc.subcore_barrier()

        for q in range(4):
            pltpu.make_async_copy(idx_hbm.at[base + q], ibufs[q],
                                  isems[q]).start()
        pltpu.make_async_copy(idx_hbm.at[base], ibufs[0], isems[0]).wait()
        pltpu.make_async_copy(h_hbm.at[ibufs[0].at[0]], gbufs[0],
                              gsems[0]).start()

        @pl.loop(0, _BPT, step=4)
        def _(g):
            for u in range(4):
                b = g + u
                p = u % 2
                pn = (u + 1) % 2
                qn = (u + 1) % 4

                @pl.when(b + 1 < _BPT)
                def _():
                    pltpu.make_async_copy(idx_hbm.at[base + b + 1],
                                          ibufs[qn], isems[qn]).wait()
                    pltpu.make_async_copy(h_hbm.at[ibufs[qn].at[0]],
                                          gbufs[pn], gsems[pn]).start()

                pltpu.make_async_copy(h_hbm.at[ibufs[u].at[0]], gbufs[p],
                                      gsems[p]).wait()
                pltpu.sync_copy(gbufs[p], acc.at[ibufs[u].at[1]], add=True)

                @pl.when(b + 4 < _BPT)
                def _():
                    pltpu.make_async_copy(idx_hbm.at[base + b + 4],
                                          ibufs[u], isems[u]).start()

        plsc.subcore_barrier()

        pltpu.sync_copy(acc.at[pl.ds(sid * _RPT, _RPT)],
                        out_hbm.at[cid, pl.ds(sid * _RPT, _RPT)])

    return k(h, idx3d)


def _tc_update(parts, h, W, b):
    rb = 2000

    def body(p_ref, h_ref, w_ref, b_ref, o_ref):
        agg = p_ref[0] + p_ref[1]
        z = jnp.dot(agg, w_ref[...], preferred_element_type=jnp.float32)
        o_ref[...] = jnp.maximum(z + b_ref[...], 0.0) + h_ref[...]

    return pl.pallas_call(
        body,
        grid=(_N // rb,),
        in_specs=[
            pl.BlockSpec((_NC, rb, _D), lambda i: (0, i, 0)),
            pl.BlockSpec((rb, _D), lambda i: (i, 0)),
            pl.BlockSpec((_D, _D), lambda i: (0, 0)),
            pl.BlockSpec((1, _D), lambda i: (0, 0)),
        ],
        out_specs=pl.BlockSpec((rb, _D), lambda i: (i, 0)),
        out_shape=jax.ShapeDtypeStruct((_N, _D), jnp.float32),
    )(parts, h, W, b)


def kernel(pcd, c_input, edge_index, W_flat, b_flat, W_fc, b_fc,
           Wg1, bg1, Wg2, bg2, Wg3, bg3):
    c2 = c_input.reshape(_N, _K * _DIN)
    h = _tc_pre(pcd, c2, W_flat, b_flat.reshape(1, _D),
                W_fc, b_fc.reshape(1, _D))

    pad = _EPAD - _E
    src = jnp.concatenate([edge_index[0], jnp.zeros((pad,), edge_index.dtype)])
    dst = jnp.concatenate([edge_index[1], jnp.full((pad,), _N, edge_index.dtype)])
    idx3d = jnp.stack([src.reshape(_NBLK, _BLK), dst.reshape(_NBLK, _BLK)],
                      axis=1)

    for Wg, bg in ((Wg1, bg1), (Wg2, bg2), (Wg3, bg3)):
        parts = _sc_segment_sum(h, idx3d)
        h = _tc_update(parts, h, Wg, bg.reshape(1, _D))

    return h.reshape(_BS, _NP, _D)
